# Initial kernel scaffold; baseline (speedup 1.0000x reference)
import jax
import jax.numpy as jnp
from jax.experimental import pallas as pl


def kernel(x, edge_index, batch, Win, bin_, convW, convB, theta, adaptW, adaptB, bnG, bnB, fc1W, fc1b, fc2W, fc2b):
    raise NotImplementedError("write your pallas kernel here")


# speedup vs baseline: 2.9823x; 2.9823x over previous
import functools

import jax
import jax.numpy as jnp
from jax import lax
from jax.experimental import pallas as pl
from jax.experimental.pallas import tpu as pltpu
from jax.experimental.pallas import tpu_sc as plsc

N = 10000
E = 320000
IN_DIM = 6
H = 128
K = 16
L = 4
G = 64

NW = 32
ROWS = 320
NP = NW * ROWS
C = 128
C0 = 1024
EP = E + C0
BR = 2048
GRID_N = NP // BR

def _wid():
    return lax.axis_index("s") * 2 + lax.axis_index("c")


def _deg_body(dsts_hbm, srcs_hbm, bnd_hbm, degd_hbm, degs_hbm,
              buf_v, cntd_v, cnts_v, bnd_v):
    wid = _wid()
    base = wid * ROWS
    pltpu.sync_copy(bnd_hbm.at[pl.ds(wid * 16, 16)], bnd_v)
    bv = bnd_v[...]
    zero16 = jnp.zeros((16,), jnp.float32)
    one16 = jnp.where(lax.iota(jnp.int32, 16) == 0, 1.0, 0.0)
    iota16 = lax.iota(jnp.int32, 16)

    def zero_body(r, _):
        cntd_v[r, :] = zero16
        cnts_v[r, :] = zero16
        return 0

    lax.fori_loop(0, ROWS + 8, zero_body, 0)

    def count(arr_hbm, cnt_v, e0, e1):
        e0a = (e0 // 8) * 8
        nch = (e1 - e0a + C0 - 1) // C0

        def chunk(ci, _):
            c0 = e0a + ci * C0
            pltpu.sync_copy(arr_hbm.at[pl.ds(c0, C0)], buf_v)

            def group(g, _):
                gb = g * 16
                v16 = buf_v[pl.ds(gb, 16)] - base
                e16 = c0 + gb + iota16
                valid = (e16 >= e0) & (e16 < e1)
                idx16 = jnp.where(valid, jnp.clip(v16, 0, ROWS - 1), ROWS)
                for i in range(16):
                    plsc.addupdate(cnt_v.at[idx16[i], :], one16)
                return 0

            lax.fori_loop(0, C0 // 16, group, 0)
            return 0

        lax.fori_loop(0, nch, chunk, 0)

    count(dsts_hbm, cntd_v, bv[0], bv[1])
    count(srcs_hbm, cnts_v, bv[2], bv[3])
    pltpu.sync_copy(cntd_v.at[pl.ds(0, ROWS)], degd_hbm.at[pl.ds(base, ROWS)])
    pltpu.sync_copy(cnts_v.at[pl.ds(0, ROWS)], degs_hbm.at[pl.ds(base, ROWS)])


def _prop_body(zt_hbm, src_hbm, dst_hbm, bnd_hbm, dd_hbm, ds_hbm,
               z_out, zt_out,
               src_v, dst_v, gat_v, acc_v, ztb_v, ddv, dsv, bnd_v, sem):
    wid = _wid()
    base = wid * ROWS
    pltpu.sync_copy(bnd_hbm.at[pl.ds(wid * 16, 16)], bnd_v)
    bv = bnd_v[...]
    e0 = bv[0]
    e1 = bv[1]
    e0a = (e0 // 8) * 8
    nch = (e1 - e0a + C - 1) // C
    zero16 = jnp.zeros((16,), jnp.float32)
    iota16 = lax.iota(jnp.int32, 16)

    def zero_body(r, _):
        for j in range(H // 16):
            acc_v[r, pl.ds(j * 16, 16)] = zero16
        return 0

    lax.fori_loop(0, ROWS + 1, zero_body, 0)

    def chunk(ci, _):
        c0 = e0a + ci * C
        pltpu.sync_copy(src_hbm.at[pl.ds(c0, C)], src_v)
        pltpu.sync_copy(dst_hbm.at[pl.ds(c0, C)], dst_v)
        pltpu.async_copy(zt_hbm.at[src_v], gat_v, sem).wait()

        def group(g, _):
            gb = g * 16
            dl16 = dst_v[pl.ds(gb, 16)] - base
            e16 = c0 + gb + iota16
            valid = (e16 >= e0) & (e16 < e1)
            idx16 = jnp.where(valid, jnp.clip(dl16, 0, ROWS - 1), ROWS)
            for i in range(16):
                dl = idx16[i]
                row = gb + i
                for j in range(H // 16):
                    plsc.addupdate(acc_v.at[dl, pl.ds(j * 16, 16)],
                                   gat_v[row, pl.ds(j * 16, 16)])
            return 0

        lax.fori_loop(0, C // 16, group, 0)
        return 0

    lax.fori_loop(0, nch, chunk, 0)

    pltpu.sync_copy(dd_hbm.at[pl.ds(base, ROWS)], ddv)
    pltpu.sync_copy(ds_hbm.at[pl.ds(base, ROWS)], dsv)

    def wb(g, _):
        gb = g * 16
        dd16 = ddv[pl.ds(gb, 16)]
        dds16 = dd16 * dsv[pl.ds(gb, 16)]
        for i in range(16):
            r = gb + i
            dd = dd16[i]
            dds = dds16[i]
            for j in range(H // 16):
                a = acc_v[r, pl.ds(j * 16, 16)]
                acc_v[r, pl.ds(j * 16, 16)] = a * dd
                ztb_v[r, pl.ds(j * 16, 16)] = a * dds
        return 0

    lax.fori_loop(0, ROWS // 16, wb, 0)
    pltpu.sync_copy(acc_v.at[pl.ds(0, ROWS)], z_out.at[pl.ds(base, ROWS)])
    pltpu.sync_copy(ztb_v, zt_out.at[pl.ds(base, ROWS)])


@functools.lru_cache(maxsize=1)
def _sc_kernels():
    mesh = plsc.VectorSubcoreMesh(core_axis_name="c", subcore_axis_name="s",
                                  num_cores=2, num_subcores=16)
    deg_kernel = pl.kernel(
        _deg_body,
        out_type=(
            jax.ShapeDtypeStruct((NP, 16), jnp.float32),
            jax.ShapeDtypeStruct((NP, 16), jnp.float32),
        ),
        mesh=mesh,
        scratch_types=[
            pltpu.VMEM((C0,), jnp.int32),
            pltpu.VMEM((ROWS + 8, 16), jnp.float32),
            pltpu.VMEM((ROWS + 8, 16), jnp.float32),
            pltpu.VMEM((16,), jnp.int32),
        ],
    )
    prop_kernel = pl.kernel(
        _prop_body,
        out_type=(
            jax.ShapeDtypeStruct((NP, H), jnp.float32),
            jax.ShapeDtypeStruct((NP, H), jnp.float32),
        ),
        mesh=mesh,
        scratch_types=[
            pltpu.VMEM((C,), jnp.int32),
            pltpu.VMEM((C,), jnp.int32),
            pltpu.VMEM((C, H), jnp.float32),
            pltpu.VMEM((ROWS + 1, H), jnp.float32),
            pltpu.VMEM((ROWS, H), jnp.float32),
            pltpu.VMEM((ROWS,), jnp.float32),
            pltpu.VMEM((ROWS,), jnp.float32),
            pltpu.VMEM((16,), jnp.int32),
            pltpu.SemaphoreType.DMA,
        ],
    )
    return deg_kernel, prop_kernel


def _row_mask(b, shape):
    rows = b * BR + lax.broadcasted_iota(jnp.int32, shape, 0)
    return rows < N


def _p1_body(degd_ref, degs_ref, dd_ref, ds_ref):
    for deg_ref, out_ref in ((degd_ref, dd_ref), (degs_ref, ds_ref)):
        deg = jnp.sum(deg_ref[...], axis=1, keepdims=True)
        out_ref[...] = jnp.where(deg > 0, lax.rsqrt(jnp.maximum(deg, 1.0)), 0.0)


def _dinv_kernel(degd, degs):
    return pl.pallas_call(
        _p1_body,
        out_shape=(jax.ShapeDtypeStruct((NP, 1), jnp.float32),
                   jax.ShapeDtypeStruct((NP, 1), jnp.float32)),
    )(degd, degs)


def _h_body(x_ref, w_ref, b_ref, dsrc_ref, h_ref, ht_ref, csum_ref):
    b = pl.program_id(0)
    hb = jnp.maximum(jnp.dot(x_ref[...], w_ref[...],
                             preferred_element_type=jnp.float32) + b_ref[...], 0.0)
    hb = jnp.where(_row_mask(b, hb.shape), hb, 0.0)
    h_ref[...] = hb
    ht_ref[...] = hb * dsrc_ref[...]

    @pl.when(b == 0)
    def _():
        csum_ref[...] = jnp.zeros_like(csum_ref)

    csum_ref[...] += jnp.sum(hb, axis=0, keepdims=True)


def _h_kernel(x_pad, win_pad, bin2, dinv_src):
    return pl.pallas_call(
        _h_body,
        grid=(GRID_N,),
        in_specs=[
            pl.BlockSpec((BR, 8), lambda b: (b, 0)),
            pl.BlockSpec((8, H), lambda b: (0, 0)),
            pl.BlockSpec((1, H), lambda b: (0, 0)),
            pl.BlockSpec((BR, 1), lambda b: (b, 0)),
        ],
        out_specs=[
            pl.BlockSpec((BR, H), lambda b: (b, 0)),
            pl.BlockSpec((BR, H), lambda b: (b, 0)),
            pl.BlockSpec((1, H), lambda b: (0, 0)),
        ],
        out_shape=(jax.ShapeDtypeStruct((NP, H), jnp.float32),
                   jax.ShapeDtypeStruct((NP, H), jnp.float32),
                   jax.ShapeDtypeStruct((1, H), jnp.float32)),
    )(x_pad, win_pad, bin2, dinv_src)


def _fw_body(csum_ref, aw_ref, ab_ref, th_ref, fw_ref):
    ctx = csum_ref[...] / jnp.float32(N)
    r = jnp.sum(aw_ref[...] * ctx, axis=1) + ab_ref[0, :] + th_ref[0, :]
    r = r - jnp.max(r)
    e = jnp.exp(r)
    fw_ref[...] = (e / jnp.sum(e))[None, :]


def _fw_kernel(csum, aw_l, ab_l, th_l):
    return pl.pallas_call(
        _fw_body,
        out_shape=jax.ShapeDtypeStruct((1, K), jnp.float32),
    )(csum, aw_l, ab_l, th_l)


def _comb_body(*refs):
    fw_ref = refs[0]
    z_refs = refs[1:1 + K]
    cw_ref, cb_ref = refs[1 + K], refs[2 + K]
    hc_ref, ssum_ref, ssq_ref = refs[3 + K:]
    b = pl.program_id(0)
    acc = fw_ref[0, 0] * z_refs[0][...]
    for k in range(1, K):
        acc += fw_ref[0, k] * z_refs[k][...]
    hc = jnp.dot(acc, cw_ref[...], preferred_element_type=jnp.float32) + cb_ref[...]
    hc = jnp.where(_row_mask(b, hc.shape), hc, 0.0)
    hc_ref[...] = hc

    @pl.when(b == 0)
    def _():
        ssum_ref[...] = jnp.zeros_like(ssum_ref)
        ssq_ref[...] = jnp.zeros_like(ssq_ref)

    ssum_ref[...] += jnp.sum(hc, axis=0, keepdims=True)
    ssq_ref[...] += jnp.sum(hc * hc, axis=0, keepdims=True)


def _comb_kernel(fw, zs, cw_l, cb_l):
    return pl.pallas_call(
        _comb_body,
        grid=(GRID_N,),
        in_specs=(
            [pl.BlockSpec((1, K), lambda b: (0, 0))]
            + [pl.BlockSpec((BR, H), lambda b: (b, 0)) for _ in range(K)]
            + [pl.BlockSpec((H, H), lambda b: (0, 0)),
               pl.BlockSpec((1, H), lambda b: (0, 0))]
        ),
        out_specs=[
            pl.BlockSpec((BR, H), lambda b: (b, 0)),
            pl.BlockSpec((1, H), lambda b: (0, 0)),
            pl.BlockSpec((1, H), lambda b: (0, 0)),
        ],
        out_shape=(jax.ShapeDtypeStruct((NP, H), jnp.float32),
                   jax.ShapeDtypeStruct((1, H), jnp.float32),
                   jax.ShapeDtypeStruct((1, H), jnp.float32)),
    )(fw, *zs, cw_l, cb_l)


def _bn_body(hc_ref, hid_ref, ssum_ref, ssq_ref, g_ref, bb_ref, dsrc_ref,
             h_ref, ht_ref, csum_ref):
    b = pl.program_id(0)
    mu = ssum_ref[...] / jnp.float32(N)
    var = ssq_ref[...] / jnp.float32(N) - mu * mu
    inv = lax.rsqrt(var + 1e-5) * g_ref[...]
    hb = jnp.maximum((hc_ref[...] - mu) * inv + bb_ref[...], 0.0) + hid_ref[...]
    hb = jnp.where(_row_mask(b, hb.shape), hb, 0.0)
    h_ref[...] = hb
    ht_ref[...] = hb * dsrc_ref[...]

    @pl.when(b == 0)
    def _():
        csum_ref[...] = jnp.zeros_like(csum_ref)

    csum_ref[...] += jnp.sum(hb, axis=0, keepdims=True)


def _bn_kernel(hc, hid, ssum, ssq, g_l, bb_l, dinv_src):
    return pl.pallas_call(
        _bn_body,
        grid=(GRID_N,),
        in_specs=[
            pl.BlockSpec((BR, H), lambda b: (b, 0)),
            pl.BlockSpec((BR, H), lambda b: (b, 0)),
            pl.BlockSpec((1, H), lambda b: (0, 0)),
            pl.BlockSpec((1, H), lambda b: (0, 0)),
            pl.BlockSpec((1, H), lambda b: (0, 0)),
            pl.BlockSpec((1, H), lambda b: (0, 0)),
            pl.BlockSpec((BR, 1), lambda b: (b, 0)),
        ],
        out_specs=[
            pl.BlockSpec((BR, H), lambda b: (b, 0)),
            pl.BlockSpec((BR, H), lambda b: (b, 0)),
            pl.BlockSpec((1, H), lambda b: (0, 0)),
        ],
        out_shape=(jax.ShapeDtypeStruct((NP, H), jnp.float32),
                   jax.ShapeDtypeStruct((NP, H), jnp.float32),
                   jax.ShapeDtypeStruct((1, H), jnp.float32)),
    )(hc, hid, ssum, ssq, g_l, bb_l, dinv_src)


def _pool_body(h_ref, batch_ref, gsum_ref, gcnt_ref):
    b = pl.program_id(0)
    gid = lax.broadcasted_iota(jnp.int32, (G, BR), 0)
    onehot = (gid == batch_ref[...]).astype(jnp.float32)

    @pl.when(b == 0)
    def _():
        gsum_ref[...] = jnp.zeros_like(gsum_ref)
        gcnt_ref[...] = jnp.zeros_like(gcnt_ref)

    gsum_ref[...] += jnp.dot(onehot, h_ref[...],
                             preferred_element_type=jnp.float32)
    gcnt_ref[...] += jnp.sum(onehot, axis=1, keepdims=True)


def _pool_kernel(h, batch_row):
    return pl.pallas_call(
        _pool_body,
        grid=(GRID_N,),
        in_specs=[
            pl.BlockSpec((BR, H), lambda b: (b, 0)),
            pl.BlockSpec((1, BR), lambda b: (0, b)),
        ],
        out_specs=[
            pl.BlockSpec((G, H), lambda b: (0, 0)),
            pl.BlockSpec((G, 1), lambda b: (0, 0)),
        ],
        out_shape=(jax.ShapeDtypeStruct((G, H), jnp.float32),
                   jax.ShapeDtypeStruct((G, 1), jnp.float32)),
    )(h, batch_row)


def _mlp_body(gsum_ref, gcnt_ref, w1_ref, b1_ref, w2_ref, b2_ref, o_ref):
    ge = gsum_ref[...] / jnp.maximum(gcnt_ref[...], 1.0)
    o = jnp.maximum(jnp.dot(ge, w1_ref[...],
                            preferred_element_type=jnp.float32) + b1_ref[...], 0.0)
    o_ref[...] = jnp.dot(o, w2_ref[...],
                         preferred_element_type=jnp.float32) + b2_ref[...]


def _mlp_kernel(gsum, gcnt, w1, b1, w2, b2):
    return pl.pallas_call(
        _mlp_body,
        out_shape=jax.ShapeDtypeStruct((G, 1), jnp.float32),
    )(gsum, gcnt, w1, b1, w2, b2)


def kernel(x, edge_index, batch, Win, bin_, convW, convB, theta, adaptW,
           adaptB, bnG, bnB, fc1W, fc1b, fc2W, fc2b):
    src, dst = edge_index[0], edge_index[1]

    order = jnp.argsort(dst)
    dst_s = dst[order]
    src_s = src[order]
    src_sorted = jnp.sort(src)
    tile_edges = jnp.arange(0, NP + 1, ROWS, dtype=jnp.int32)
    be = jnp.searchsorted(dst_s, tile_edges).astype(jnp.int32)
    bs = jnp.searchsorted(src_sorted, tile_edges).astype(jnp.int32)
    bnd = jnp.zeros((NW, 16), jnp.int32)
    bnd = (bnd.at[:, 0].set(be[:-1]).at[:, 1].set(be[1:])
              .at[:, 2].set(bs[:-1]).at[:, 3].set(bs[1:])).reshape(NW * 16)
    dst_sp = jnp.pad(dst_s, (0, EP - E))
    src_sp = jnp.pad(src_s, (0, EP - E))
    srcs_p = jnp.pad(src_sorted, (0, EP - E))

    x_pad = jnp.zeros((NP, 8), jnp.float32).at[:N, :IN_DIM].set(x)
    win_pad = jnp.zeros((8, H), jnp.float32).at[:IN_DIM, :].set(Win)
    batch_row = jnp.full((1, NP), G, jnp.int32).at[0, :N].set(batch)

    _deg_kernel, _prop_kernel = _sc_kernels()
    degd2, degs2 = _deg_kernel(dst_sp, srcs_p, bnd)
    dinv_dst, dinv_src = _dinv_kernel(degd2, degs2)
    dd_flat = dinv_dst.reshape(NP)
    ds_flat = dinv_src.reshape(NP)

    h, ht, csum = _h_kernel(x_pad, win_pad, bin_.reshape(1, H), dinv_src)

    all_fw = []
    for l in range(L):
        fw = _fw_kernel(csum, adaptW[l], adaptB[l].reshape(1, K),
                        theta[l].reshape(1, K))
        all_fw.append(fw)
        zs = [h]
        zt = ht
        for _ in range(K - 1):
            z_new, zt = _prop_kernel(zt, src_sp, dst_sp, bnd, dd_flat,
                                     ds_flat)
            zs.append(z_new)
        hc, ssum, ssq = _comb_kernel(fw, zs, convW[l], convB[l].reshape(1, H))
        h, ht, csum = _bn_kernel(hc, h, ssum, ssq, bnG[l].reshape(1, H),
                                 bnB[l].reshape(1, H), dinv_src)

    gsum, gcnt = _pool_kernel(h, batch_row)
    o = _mlp_kernel(gsum, gcnt, fc1W, fc1b.reshape(1, H // 2), fc2W,
                    fc2b.reshape(1, 1))
    return o, jnp.concatenate(all_fw, axis=0)
